# baseline (speedup 1.0000x reference)
import functools

import jax
import jax.numpy as jnp
from jax import lax
from jax.experimental import pallas as pl
from jax.experimental.pallas import tpu as pltpu
from jax.experimental.pallas import tpu_sc as plsc

_SWAPS = 64
_SEED = 42
_B = 16
_TOTAL = 32768
_D = 256
_NFIX = _B * _SWAPS * 2

_NUM_SC = 2
_NUM_SUBCORES = 16
_NW = _NUM_SC * _NUM_SUBCORES
_ROWS_PER_W = _TOTAL // _NW
_GCHUNK = 128


def _threefry_core(k1, k2, x0, x1):
    ks2 = k1 ^ k2 ^ 0x1BD11BDA

    def rotl(x, r):
        return (x << r) | lax.shift_right_logical(x, 32 - r)

    x0 = x0 + k1
    x1 = x1 + k2
    sched = (
        ((13, 15, 26, 6), k2, ks2, 1),
        ((17, 29, 16, 24), ks2, k1, 2),
        ((13, 15, 26, 6), k1, k2, 3),
        ((17, 29, 16, 24), k2, ks2, 4),
        ((13, 15, 26, 6), ks2, k1, 5),
    )
    for rots, ka, kb, c in sched:
        for r in rots:
            x0 = x0 + x1
            x1 = rotl(x1, r)
            x1 = x0 ^ x1
        x0 = x0 + ka
        x1 = x1 + kb + c
    return x0, x1


def _fixes_body(cu_ref, dst_ref, src_ref):
    shape = (_B, 2 * _SWAPS)
    lane = lax.broadcasted_iota(jnp.int32, shape, 1)
    row = lax.broadcasted_iota(jnp.int32, shape, 0)
    s = jnp.zeros(shape, jnp.int32)
    e = jnp.zeros(shape, jnp.int32)
    for i in range(_B):
        s = jnp.where(row == i, cu_ref[i], s)
        e = jnp.where(row == i, cu_ref[i + 1], e)
    n = e - s
    t = row * _SWAPS + lane // 2
    j = lane & 1

    zero = jnp.zeros(shape, jnp.int32)
    k1, k2 = _threefry_core(zero, zero + _SEED, zero, t)
    a1, a2 = _threefry_core(k1, k2, zero, zero)
    b1, b2 = _threefry_core(k1, k2, zero, zero + 1)
    h1, h2 = _threefry_core(a1, a2, zero, j)
    hbits = h1 ^ h2
    l1, l2 = _threefry_core(b1, b2, zero, j)
    lbits = l1 ^ l2

    span = jnp.maximum(n, 1)
    span_f = span.astype(jnp.float32)

    def umod24(x):
        q = jnp.floor(x.astype(jnp.float32) / span_f).astype(jnp.int32)
        r = x - q * span
        for _ in range(3):
            r = jnp.where(r < 0, r + span, r)
        for _ in range(3):
            r = jnp.where(r >= span, r - span, r)
        return r

    def umod31(x):
        q1 = jnp.floor(x.astype(jnp.float32) / span_f).astype(jnp.int32)
        return umod24(x - q1 * span + 140 * span)

    m16 = umod24(zero + 65536)

    def umod32(x):
        return umod31(lax.shift_right_logical(x, 16) * m16 + (x & 0xFFFF))

    mult = umod31(m16 * m16)
    off = umod31(umod32(hbits) * mult + umod32(lbits))

    idx = jnp.where(n > 1, off, 0)
    gpos = s + idx

    val = gpos
    for tt in range(_SWAPS):
        g1 = gpos[:, 2 * tt:2 * tt + 1]
        g2 = gpos[:, 2 * tt + 1:2 * tt + 2]
        v1 = val[:, 2 * tt:2 * tt + 1]
        v2 = val[:, 2 * tt + 1:2 * tt + 2]
        val = jnp.where(gpos == g1, v2, jnp.where(gpos == g2, v1, val))
    dst_ref[:, :] = gpos
    src_ref[:, :] = val


_fixes_call = pl.pallas_call(
    _fixes_body,
    in_specs=[pl.BlockSpec(memory_space=pltpu.SMEM)],
    out_specs=(
        pl.BlockSpec(memory_space=pltpu.VMEM),
        pl.BlockSpec(memory_space=pltpu.VMEM),
    ),
    out_shape=(
        jax.ShapeDtypeStruct((_B, 2 * _SWAPS), jnp.int32),
        jax.ShapeDtypeStruct((_B, 2 * _SWAPS), jnp.int32),
    ),
)


_ROWS_PER_SC = _TOTAL // _NUM_SC
_FIX_PER_TILE = _NFIX // _NUM_SUBCORES


_NBUF = 3


def _sc_apply_body(flat_hbm, iota_hbm, dst_hbm, src_hbm, out_hbm,
                   perm_sh, pidx_v, dfix_v, sfix_v, rows_bufs, gsems, ssems):
    cid = lax.axis_index("c")
    sid = lax.axis_index("s")
    wid = cid * _NUM_SUBCORES + sid
    base = pl.multiple_of(wid * _ROWS_PER_W, _ROWS_PER_W)
    sc_lo = pl.multiple_of(cid * _ROWS_PER_SC, _ROWS_PER_SC)

    @pl.when(sid == 0)
    def _init():
        pltpu.sync_copy(iota_hbm.at[pl.ds(sc_lo, _ROWS_PER_SC)],
                        perm_sh.at[pl.ds(0, _ROWS_PER_SC)])

    foff = pl.multiple_of(sid * _FIX_PER_TILE, _FIX_PER_TILE)
    pltpu.sync_copy(dst_hbm.at[pl.ds(foff, _FIX_PER_TILE)], dfix_v)
    pltpu.sync_copy(src_hbm.at[pl.ds(foff, _FIX_PER_TILE)], sfix_v)

    sink0 = _ROWS_PER_SC + sid * _FIX_PER_TILE
    lane16 = lax.iota(jnp.int32, 16)
    for k in range(_FIX_PER_TILE // 16):
        d16 = dfix_v[pl.ds(k * 16, 16)]
        s16 = sfix_v[pl.ds(k * 16, 16)]
        inr = (d16 >= sc_lo) & (d16 < sc_lo + _ROWS_PER_SC) & (d16 != s16)
        dfix_v[pl.ds(k * 16, 16)] = jnp.where(
            inr, d16 - sc_lo, sink0 + k * 16 + lane16)

    plsc.subcore_barrier()
    pltpu.sync_copy(sfix_v, perm_sh.at[dfix_v])
    plsc.subcore_barrier()

    loc = pl.multiple_of(sid * _ROWS_PER_W, _ROWS_PER_W)
    pltpu.sync_copy(perm_sh.at[pl.ds(loc, _ROWS_PER_W)], pidx_v)

    nch = _ROWS_PER_W // _GCHUNK

    def gather_start(c, b):
        return pltpu.async_copy(
            flat_hbm.at[pidx_v.at[pl.ds(c * _GCHUNK, _GCHUNK)]],
            rows_bufs[b], gsems[b])

    def store_start(c, b):
        return pltpu.async_copy(
            rows_bufs[b], out_hbm.at[pl.ds(base + c * _GCHUNK, _GCHUNK)],
            ssems[b])

    g = [gather_start(b, b) for b in range(_NBUF)]
    s = [None] * _NBUF
    for c in range(nch):
        b = c % _NBUF
        g[b].wait()
        s[b] = store_start(c, b)
        if c + _NBUF < nch:
            s[b].wait()
            g[b] = gather_start(c + _NBUF, b)
    for c in range(max(0, nch - _NBUF), nch):
        s[c % _NBUF].wait()


@functools.cache
def _make_sc_apply():
    mesh = plsc.VectorSubcoreMesh(
        core_axis_name="c", subcore_axis_name="s",
        num_cores=_NUM_SC, num_subcores=_NUM_SUBCORES,
    )
    return functools.partial(
        pl.kernel,
        mesh=mesh,
        out_type=jax.ShapeDtypeStruct((_TOTAL, _D), jnp.float32),
        scratch_types=[
            pltpu.VMEM_SHARED((_ROWS_PER_SC + _NFIX,), jnp.int32),
            pltpu.VMEM((_ROWS_PER_W,), jnp.int32),
            pltpu.VMEM((_FIX_PER_TILE,), jnp.int32),
            pltpu.VMEM((_FIX_PER_TILE,), jnp.int32),
            [pltpu.VMEM((_GCHUNK, _D), jnp.float32) for _ in range(_NBUF)],
            [pltpu.SemaphoreType.DMA for _ in range(_NBUF)],
            [pltpu.SemaphoreType.DMA for _ in range(_NBUF)],
        ],
    )(_sc_apply_body)


def kernel(flat, cu_seqlens):
    cu = cu_seqlens.astype(jnp.int32)
    dst, src = _fixes_call(cu)
    iota = jnp.arange(_TOTAL, dtype=jnp.int32)
    return _make_sc_apply()(flat, iota, dst.reshape(_NFIX), src.reshape(_NFIX))


# speedup vs baseline: 1.0131x; 1.0131x over previous
import functools

import jax
import jax.numpy as jnp
from jax import lax
from jax.experimental import pallas as pl
from jax.experimental.pallas import tpu as pltpu
from jax.experimental.pallas import tpu_sc as plsc

_SWAPS = 64
_SEED = 42
_B = 16
_TOTAL = 32768
_D = 256
_NFIX = _B * _SWAPS * 2

_NUM_SC = 2
_NUM_SUBCORES = 16
_NW = _NUM_SC * _NUM_SUBCORES
_ROWS_PER_W = _TOTAL // _NW
_GCHUNK = 64


def _threefry_core(k1, k2, x0, x1):
    ks2 = k1 ^ k2 ^ 0x1BD11BDA

    def rotl(x, r):
        return (x << r) | lax.shift_right_logical(x, 32 - r)

    x0 = x0 + k1
    x1 = x1 + k2
    sched = (
        ((13, 15, 26, 6), k2, ks2, 1),
        ((17, 29, 16, 24), ks2, k1, 2),
        ((13, 15, 26, 6), k1, k2, 3),
        ((17, 29, 16, 24), k2, ks2, 4),
        ((13, 15, 26, 6), ks2, k1, 5),
    )
    for rots, ka, kb, c in sched:
        for r in rots:
            x0 = x0 + x1
            x1 = rotl(x1, r)
            x1 = x0 ^ x1
        x0 = x0 + ka
        x1 = x1 + kb + c
    return x0, x1


def _fixes_body(cu_ref, dst_ref, src_ref):
    shape = (_B, 2 * _SWAPS)
    lane = lax.broadcasted_iota(jnp.int32, shape, 1)
    row = lax.broadcasted_iota(jnp.int32, shape, 0)
    s = jnp.zeros(shape, jnp.int32)
    e = jnp.zeros(shape, jnp.int32)
    for i in range(_B):
        s = jnp.where(row == i, cu_ref[i], s)
        e = jnp.where(row == i, cu_ref[i + 1], e)
    n = e - s
    t = row * _SWAPS + lane // 2
    j = lane & 1

    zero = jnp.zeros(shape, jnp.int32)
    k1, k2 = _threefry_core(zero, zero + _SEED, zero, t)
    a1, a2 = _threefry_core(k1, k2, zero, zero)
    b1, b2 = _threefry_core(k1, k2, zero, zero + 1)
    h1, h2 = _threefry_core(a1, a2, zero, j)
    hbits = h1 ^ h2
    l1, l2 = _threefry_core(b1, b2, zero, j)
    lbits = l1 ^ l2

    span = jnp.maximum(n, 1)
    span_f = span.astype(jnp.float32)

    def umod24(x):
        q = jnp.floor(x.astype(jnp.float32) / span_f).astype(jnp.int32)
        r = x - q * span
        for _ in range(3):
            r = jnp.where(r < 0, r + span, r)
        for _ in range(3):
            r = jnp.where(r >= span, r - span, r)
        return r

    def umod31(x):
        q1 = jnp.floor(x.astype(jnp.float32) / span_f).astype(jnp.int32)
        return umod24(x - q1 * span + 140 * span)

    m16 = umod24(zero + 65536)

    def umod32(x):
        return umod31(lax.shift_right_logical(x, 16) * m16 + (x & 0xFFFF))

    mult = umod31(m16 * m16)
    off = umod31(umod32(hbits) * mult + umod32(lbits))

    idx = jnp.where(n > 1, off, 0)
    gpos = s + idx

    val = gpos
    for tt in range(_SWAPS):
        g1 = gpos[:, 2 * tt:2 * tt + 1]
        g2 = gpos[:, 2 * tt + 1:2 * tt + 2]
        v1 = val[:, 2 * tt:2 * tt + 1]
        v2 = val[:, 2 * tt + 1:2 * tt + 2]
        val = jnp.where(gpos == g1, v2, jnp.where(gpos == g2, v1, val))
    dst_ref[:, :] = gpos
    src_ref[:, :] = val


_fixes_call = pl.pallas_call(
    _fixes_body,
    in_specs=[pl.BlockSpec(memory_space=pltpu.SMEM)],
    out_specs=(
        pl.BlockSpec(memory_space=pltpu.VMEM),
        pl.BlockSpec(memory_space=pltpu.VMEM),
    ),
    out_shape=(
        jax.ShapeDtypeStruct((_B, 2 * _SWAPS), jnp.int32),
        jax.ShapeDtypeStruct((_B, 2 * _SWAPS), jnp.int32),
    ),
)


_ROWS_PER_SC = _TOTAL // _NUM_SC
_FIX_PER_TILE = _NFIX // _NUM_SUBCORES


_NBUF = 6


def _sc_apply_body(flat_hbm, iota_hbm, dst_hbm, src_hbm, out_hbm,
                   perm_sh, pidx_v, dfix_v, sfix_v, rows_bufs, gsems, ssems):
    cid = lax.axis_index("c")
    sid = lax.axis_index("s")
    wid = cid * _NUM_SUBCORES + sid
    base = pl.multiple_of(wid * _ROWS_PER_W, _ROWS_PER_W)
    sc_lo = pl.multiple_of(cid * _ROWS_PER_SC, _ROWS_PER_SC)

    loc = pl.multiple_of(sid * _ROWS_PER_W, _ROWS_PER_W)
    pltpu.sync_copy(iota_hbm.at[pl.ds(sc_lo + loc, _ROWS_PER_W)],
                    perm_sh.at[pl.ds(loc, _ROWS_PER_W)])

    foff = pl.multiple_of(sid * _FIX_PER_TILE, _FIX_PER_TILE)
    pltpu.sync_copy(dst_hbm.at[pl.ds(foff, _FIX_PER_TILE)], dfix_v)
    pltpu.sync_copy(src_hbm.at[pl.ds(foff, _FIX_PER_TILE)], sfix_v)

    sink0 = _ROWS_PER_SC + sid * _FIX_PER_TILE
    lane16 = lax.iota(jnp.int32, 16)
    for k in range(_FIX_PER_TILE // 16):
        d16 = dfix_v[pl.ds(k * 16, 16)]
        s16 = sfix_v[pl.ds(k * 16, 16)]
        inr = (d16 >= sc_lo) & (d16 < sc_lo + _ROWS_PER_SC) & (d16 != s16)
        dfix_v[pl.ds(k * 16, 16)] = jnp.where(
            inr, d16 - sc_lo, sink0 + k * 16 + lane16)

    plsc.subcore_barrier()
    pltpu.sync_copy(sfix_v, perm_sh.at[dfix_v])
    plsc.subcore_barrier()

    pltpu.sync_copy(perm_sh.at[pl.ds(loc, _ROWS_PER_W)], pidx_v)

    nch = _ROWS_PER_W // _GCHUNK

    def gather_start(c, b):
        return pltpu.async_copy(
            flat_hbm.at[pidx_v.at[pl.ds(c * _GCHUNK, _GCHUNK)]],
            rows_bufs[b], gsems[b])

    def store_start(c, b):
        return pltpu.async_copy(
            rows_bufs[b], out_hbm.at[pl.ds(base + c * _GCHUNK, _GCHUNK)],
            ssems[b])

    g = [gather_start(b, b) for b in range(_NBUF)]
    s = [None] * _NBUF
    for c in range(nch):
        b = c % _NBUF
        g[b].wait()
        s[b] = store_start(c, b)
        if c + _NBUF < nch:
            s[b].wait()
            g[b] = gather_start(c + _NBUF, b)
    for c in range(max(0, nch - _NBUF), nch):
        s[c % _NBUF].wait()


@functools.cache
def _make_sc_apply():
    mesh = plsc.VectorSubcoreMesh(
        core_axis_name="c", subcore_axis_name="s",
        num_cores=_NUM_SC, num_subcores=_NUM_SUBCORES,
    )
    return functools.partial(
        pl.kernel,
        mesh=mesh,
        out_type=jax.ShapeDtypeStruct((_TOTAL, _D), jnp.float32),
        scratch_types=[
            pltpu.VMEM_SHARED((_ROWS_PER_SC + _NFIX,), jnp.int32),
            pltpu.VMEM((_ROWS_PER_W,), jnp.int32),
            pltpu.VMEM((_FIX_PER_TILE,), jnp.int32),
            pltpu.VMEM((_FIX_PER_TILE,), jnp.int32),
            [pltpu.VMEM((_GCHUNK, _D), jnp.float32) for _ in range(_NBUF)],
            [pltpu.SemaphoreType.DMA for _ in range(_NBUF)],
            [pltpu.SemaphoreType.DMA for _ in range(_NBUF)],
        ],
    )(_sc_apply_body)


def kernel(flat, cu_seqlens):
    cu = cu_seqlens.astype(jnp.int32)
    dst, src = _fixes_call(cu)
    iota = jnp.arange(_TOTAL, dtype=jnp.int32)
    return _make_sc_apply()(flat, iota, dst.reshape(_NFIX), src.reshape(_NFIX))


# speedup vs baseline: 1.0161x; 1.0030x over previous
import functools

import jax
import jax.numpy as jnp
from jax import lax
from jax.experimental import pallas as pl
from jax.experimental.pallas import tpu as pltpu
from jax.experimental.pallas import tpu_sc as plsc

_SWAPS = 64
_SEED = 42
_B = 16
_TOTAL = 32768
_D = 256
_NFIX = _B * _SWAPS * 2

_NUM_SC = 2
_NUM_SUBCORES = 16
_NW = _NUM_SC * _NUM_SUBCORES
_ROWS_PER_W = _TOTAL // _NW
_GCHUNK = 64


def _threefry_core(k1, k2, x0, x1):
    ks2 = k1 ^ k2 ^ 0x1BD11BDA

    def rotl(x, r):
        return (x << r) | lax.shift_right_logical(x, 32 - r)

    x0 = x0 + k1
    x1 = x1 + k2
    sched = (
        ((13, 15, 26, 6), k2, ks2, 1),
        ((17, 29, 16, 24), ks2, k1, 2),
        ((13, 15, 26, 6), k1, k2, 3),
        ((17, 29, 16, 24), k2, ks2, 4),
        ((13, 15, 26, 6), ks2, k1, 5),
    )
    for rots, ka, kb, c in sched:
        for r in rots:
            x0 = x0 + x1
            x1 = rotl(x1, r)
            x1 = x0 ^ x1
        x0 = x0 + ka
        x1 = x1 + kb + c
    return x0, x1


def _fixes_body(cu_ref, dst_ref, src_ref):
    shape = (_B, 2 * _SWAPS)
    lane = lax.broadcasted_iota(jnp.int32, shape, 1)
    row = lax.broadcasted_iota(jnp.int32, shape, 0)
    s = jnp.zeros(shape, jnp.int32)
    e = jnp.zeros(shape, jnp.int32)
    for i in range(_B):
        s = jnp.where(row == i, cu_ref[i], s)
        e = jnp.where(row == i, cu_ref[i + 1], e)
    n = e - s
    t = row * _SWAPS + lane // 2
    j = lane & 1

    zero = jnp.zeros(shape, jnp.int32)
    k1, k2 = _threefry_core(zero, zero + _SEED, zero, t)
    a1, a2 = _threefry_core(k1, k2, zero, zero)
    b1, b2 = _threefry_core(k1, k2, zero, zero + 1)
    h1, h2 = _threefry_core(a1, a2, zero, j)
    hbits = h1 ^ h2
    l1, l2 = _threefry_core(b1, b2, zero, j)
    lbits = l1 ^ l2

    span = jnp.maximum(n, 1)
    span_f = span.astype(jnp.float32)

    def umod24(x):
        q = jnp.floor(x.astype(jnp.float32) / span_f).astype(jnp.int32)
        r = x - q * span
        for _ in range(3):
            r = jnp.where(r < 0, r + span, r)
        for _ in range(3):
            r = jnp.where(r >= span, r - span, r)
        return r

    def umod31(x):
        q1 = jnp.floor(x.astype(jnp.float32) / span_f).astype(jnp.int32)
        return umod24(x - q1 * span + 140 * span)

    m16 = umod24(zero + 65536)

    def umod32(x):
        return umod31(lax.shift_right_logical(x, 16) * m16 + (x & 0xFFFF))

    mult = umod31(m16 * m16)
    off = umod31(umod32(hbits) * mult + umod32(lbits))

    idx = jnp.where(n > 1, off, 0)
    gpos = s + idx

    val = gpos
    for tt in range(0, _SWAPS, 2):
        g1 = gpos[:, 2 * tt:2 * tt + 1]
        g2 = gpos[:, 2 * tt + 1:2 * tt + 2]
        g3 = gpos[:, 2 * tt + 2:2 * tt + 3]
        g4 = gpos[:, 2 * tt + 3:2 * tt + 4]
        v1 = val[:, 2 * tt:2 * tt + 1]
        v2 = val[:, 2 * tt + 1:2 * tt + 2]
        v3 = val[:, 2 * tt + 2:2 * tt + 3]
        v4 = val[:, 2 * tt + 3:2 * tt + 4]
        v3 = jnp.where(g3 == g1, v2, jnp.where(g3 == g2, v1, v3))
        v4 = jnp.where(g4 == g1, v2, jnp.where(g4 == g2, v1, v4))
        val = jnp.where(gpos == g3, v4, jnp.where(gpos == g4, v3,
              jnp.where(gpos == g1, v2, jnp.where(gpos == g2, v1, val))))
    dst_ref[:, :] = gpos
    src_ref[:, :] = val


_fixes_call = pl.pallas_call(
    _fixes_body,
    in_specs=[pl.BlockSpec(memory_space=pltpu.SMEM)],
    out_specs=(
        pl.BlockSpec(memory_space=pltpu.VMEM),
        pl.BlockSpec(memory_space=pltpu.VMEM),
    ),
    out_shape=(
        jax.ShapeDtypeStruct((_B, 2 * _SWAPS), jnp.int32),
        jax.ShapeDtypeStruct((_B, 2 * _SWAPS), jnp.int32),
    ),
)


_ROWS_PER_SC = _TOTAL // _NUM_SC
_FIX_PER_TILE = _NFIX // _NUM_SUBCORES


_NBUF = 6


def _sc_apply_body(flat_hbm, iota_hbm, dst_hbm, src_hbm, out_hbm,
                   perm_sh, pidx_v, dfix_v, sfix_v, rows_bufs, gsems, ssems):
    cid = lax.axis_index("c")
    sid = lax.axis_index("s")
    wid = cid * _NUM_SUBCORES + sid
    base = pl.multiple_of(wid * _ROWS_PER_W, _ROWS_PER_W)
    sc_lo = pl.multiple_of(cid * _ROWS_PER_SC, _ROWS_PER_SC)

    loc = pl.multiple_of(sid * _ROWS_PER_W, _ROWS_PER_W)
    pltpu.sync_copy(iota_hbm.at[pl.ds(sc_lo + loc, _ROWS_PER_W)],
                    perm_sh.at[pl.ds(loc, _ROWS_PER_W)])

    foff = pl.multiple_of(sid * _FIX_PER_TILE, _FIX_PER_TILE)
    pltpu.sync_copy(dst_hbm.at[pl.ds(foff, _FIX_PER_TILE)], dfix_v)
    pltpu.sync_copy(src_hbm.at[pl.ds(foff, _FIX_PER_TILE)], sfix_v)

    sink0 = _ROWS_PER_SC + sid * _FIX_PER_TILE
    lane16 = lax.iota(jnp.int32, 16)
    for k in range(_FIX_PER_TILE // 16):
        d16 = dfix_v[pl.ds(k * 16, 16)]
        s16 = sfix_v[pl.ds(k * 16, 16)]
        inr = (d16 >= sc_lo) & (d16 < sc_lo + _ROWS_PER_SC) & (d16 != s16)
        dfix_v[pl.ds(k * 16, 16)] = jnp.where(
            inr, d16 - sc_lo, sink0 + k * 16 + lane16)

    plsc.subcore_barrier()
    pltpu.sync_copy(sfix_v, perm_sh.at[dfix_v])
    plsc.subcore_barrier()

    pltpu.sync_copy(perm_sh.at[pl.ds(loc, _ROWS_PER_W)], pidx_v)

    nch = _ROWS_PER_W // _GCHUNK

    def gather_start(c, b):
        return pltpu.async_copy(
            flat_hbm.at[pidx_v.at[pl.ds(c * _GCHUNK, _GCHUNK)]],
            rows_bufs[b], gsems[b])

    def store_start(c, b):
        return pltpu.async_copy(
            rows_bufs[b], out_hbm.at[pl.ds(base + c * _GCHUNK, _GCHUNK)],
            ssems[b])

    g = [gather_start(b, b) for b in range(_NBUF)]
    s = [None] * _NBUF
    for c in range(nch):
        b = c % _NBUF
        g[b].wait()
        s[b] = store_start(c, b)
        if c + _NBUF < nch:
            s[b].wait()
            g[b] = gather_start(c + _NBUF, b)
    for c in range(max(0, nch - _NBUF), nch):
        s[c % _NBUF].wait()


@functools.cache
def _make_sc_apply():
    mesh = plsc.VectorSubcoreMesh(
        core_axis_name="c", subcore_axis_name="s",
        num_cores=_NUM_SC, num_subcores=_NUM_SUBCORES,
    )
    return functools.partial(
        pl.kernel,
        mesh=mesh,
        out_type=jax.ShapeDtypeStruct((_TOTAL, _D), jnp.float32),
        scratch_types=[
            pltpu.VMEM_SHARED((_ROWS_PER_SC + _NFIX,), jnp.int32),
            pltpu.VMEM((_ROWS_PER_W,), jnp.int32),
            pltpu.VMEM((_FIX_PER_TILE,), jnp.int32),
            pltpu.VMEM((_FIX_PER_TILE,), jnp.int32),
            [pltpu.VMEM((_GCHUNK, _D), jnp.float32) for _ in range(_NBUF)],
            [pltpu.SemaphoreType.DMA for _ in range(_NBUF)],
            [pltpu.SemaphoreType.DMA for _ in range(_NBUF)],
        ],
    )(_sc_apply_body)


def kernel(flat, cu_seqlens):
    cu = cu_seqlens.astype(jnp.int32)
    dst, src = _fixes_call(cu)
    iota = jnp.arange(_TOTAL, dtype=jnp.int32)
    return _make_sc_apply()(flat, iota, dst.reshape(_NFIX), src.reshape(_NFIX))
